# baseline (speedup 1.0000x reference)
import jax
import jax.numpy as jnp
from jax.experimental import pallas as pl

B, C, H, W_IMG = 16, 3, 512, 512
NUM_CLASSES, TOPK_ANCH, KEEP = 21, 200, 100
HW = H * W_IMG
ROWS = B * C
RG = 8
NSEL = (NUM_CLASSES - 1) * TOPK_ANCH
CPAD = 1024
KPAD = 256


def _feat_body(x_ref, p_ref, o_ref):
    xb = x_ref[...].reshape(RG, HW)
    o_ref[...] = jax.lax.dot_general(
        xb, p_ref[...],
        dimension_numbers=(((1,), (1,)), ((), ())),
        preferred_element_type=jnp.float32)


def _feat_pallas(x3, P):
    return pl.pallas_call(
        _feat_body,
        grid=(ROWS // RG,),
        in_specs=[
            pl.BlockSpec((RG, H, W_IMG), lambda i: (i, 0, 0)),
            pl.BlockSpec((2, HW), lambda i: (0, 0)),
        ],
        out_specs=pl.BlockSpec((RG, 2), lambda i: (i, 0)),
        out_shape=jax.ShapeDtypeStruct((ROWS, 2), jnp.float32),
    )(x3, P)


def _headtopk_body(f1_ref, f2_ref, w_ref, b_ref, sc_ref, lb_ref, bs_ref):
    f1 = f1_ref[...]
    f2 = f2_ref[...]
    w = w_ref[...]
    bias = b_ref[...]

    a1 = jax.lax.dot_general(f1, w, (((1,), (0,)), ((), ())),
                             preferred_element_type=jnp.float32) + bias
    a2 = jax.lax.dot_general(f2, w, (((1,), (0,)), ((), ())),
                             preferred_element_type=jnp.float32) + bias
    d = 0.5 * (jnp.tanh(a1) + jnp.tanh(a2))

    io0 = jax.lax.broadcasted_iota(jnp.int32, (CPAD, KPAD), 0)
    io1 = jax.lax.broadcasted_iota(jnp.int32, (CPAD, KPAD), 1)
    Es = [jnp.where((io0 == 5 * io1 + f) & (io1 < TOPK_ANCH), 1.0, 0.0)
          for f in range(5)]

    def select(field):
        parts = [jax.lax.dot_general(
            d[:, cls * CPAD:(cls + 1) * CPAD], Es[field],
            (((1,), (0,)), ((), ())),
            precision=jax.lax.Precision.HIGHEST,
            preferred_element_type=jnp.float32)
            for cls in range(NUM_CLASSES)]
        return jnp.concatenate(parts, axis=1)

    NFULL = NUM_CLASSES * KPAD
    iota_l = jax.lax.broadcasted_iota(jnp.int32, (B, NFULL), 1)
    live = (jax.lax.rem(iota_l, KPAD) < TOPK_ANCH) & (iota_l >= KPAD)
    conf0 = jnp.where(live, jax.nn.sigmoid(select(0)), -1.0)
    bp = [select(1 + j) * 512.0 for j in range(4)]

    slot_iota = jax.lax.broadcasted_iota(jnp.int32, (B, 128), 1)
    zf = jnp.zeros((B, 128), jnp.float32)
    zi = jnp.zeros((B, 128), jnp.int32)

    def step(k, carry):
        conf, sc, ix, b0, b1, b2, b3 = carry
        m = jnp.max(conf, axis=1, keepdims=True)
        sel = jnp.min(jnp.where(conf == m, iota_l, NFULL),
                      axis=1, keepdims=True)
        selmask = iota_l == sel
        oh = slot_iota == k
        sc = jnp.where(oh, m, sc)
        ix = jnp.where(oh, sel, ix)
        gath = [jnp.sum(jnp.where(selmask, p, 0.0), axis=1, keepdims=True)
                for p in bp]
        b0 = jnp.where(oh, gath[0], b0)
        b1 = jnp.where(oh, gath[1], b1)
        b2 = jnp.where(oh, gath[2], b2)
        b3 = jnp.where(oh, gath[3], b3)
        conf = jnp.where(selmask, -1.0, conf)
        return conf, sc, ix, b0, b1, b2, b3

    _, sc, ix, b0, b1, b2, b3 = jax.lax.fori_loop(
        0, KEEP, step, (conf0, zf, zi, zf, zf, zf, zf))
    sc_ref[...] = sc[:, :KEEP]
    lb_ref[...] = ix[:, :KEEP] // KPAD - 1
    bs_ref[0] = b0[:, :KEEP]
    bs_ref[1] = b1[:, :KEEP]
    bs_ref[2] = b2[:, :KEEP]
    bs_ref[3] = b3[:, :KEEP]


def _headtopk_pallas(f1, f2, W, b2):
    return pl.pallas_call(
        _headtopk_body,
        out_shape=(
            jax.ShapeDtypeStruct((B, KEEP), jnp.float32),
            jax.ShapeDtypeStruct((B, KEEP), jnp.int32),
            jax.ShapeDtypeStruct((4, B, KEEP), jnp.float32),
        ),
    )(f1, f2, W, b2)


def kernel(x, pos, W, b):
    p1 = pos.reshape(HW)
    p2 = pos[::-1, :].reshape(HW)
    P = jnp.stack([p1, p2], axis=0)
    x3 = x.reshape(ROWS, H, W_IMG)
    fp = _feat_pallas(x3, P)
    f1 = fp[:, 0].reshape(B, C)
    f2 = fp[:, 1].reshape(B, C)

    Wp = jnp.pad(W.reshape(C, NUM_CLASSES, 1000),
                 ((0, 0), (0, 0), (0, CPAD - 1000))).reshape(C, -1)
    bp_ = jnp.pad(b.reshape(NUM_CLASSES, 1000),
                  ((0, 0), (0, CPAD - 1000))).reshape(1, -1)
    top_scores, labels, bsel = _headtopk_pallas(f1, f2, Wp, bp_)
    sel_boxes = jnp.transpose(bsel, (1, 2, 0))
    return sel_boxes, top_scores, labels


# speedup vs baseline: 1.2321x; 1.2321x over previous
import functools

import jax
import jax.numpy as jnp
from jax import lax
from jax.experimental import pallas as pl
from jax.experimental.pallas import tpu as pltpu, tpu_sc as plsc

B, C, H, W_IMG = 16, 3, 512, 512
NUM_CLASSES, TOPK_ANCH, KEEP = 21, 200, 100
HW = H * W_IMG
ROWS = B * C
RG = 8
NSEL = (NUM_CLASSES - 1) * TOPK_ANCH
KPADDED = 112


def _feat_body(x_ref, p_ref, o_ref):
    xb = x_ref[...].reshape(RG, HW)
    o_ref[...] = jax.lax.dot_general(
        xb, p_ref[...],
        dimension_numbers=(((1,), (1,)), ((), ())),
        preferred_element_type=jnp.float32)


def _feat_pallas(x3, P):
    return pl.pallas_call(
        _feat_body,
        grid=(ROWS // RG,),
        in_specs=[
            pl.BlockSpec((RG, H, W_IMG), lambda i: (i, 0, 0)),
            pl.BlockSpec((2, HW), lambda i: (0, 0)),
        ],
        out_specs=pl.BlockSpec((RG, 2), lambda i: (i, 0)),
        out_shape=jax.ShapeDtypeStruct((ROWS, 2), jnp.float32),
    )(x3, P)


def _headtopk_body(f1_ref, f2_ref, wc_ref, bc_ref, wb_ref, bb_ref,
                   sc_ref, lb_ref, ix_ref, bp_ref):
    f1 = f1_ref[...]
    f2 = f2_ref[...]

    def combo(w, bias):
        a1 = jax.lax.dot_general(f1, w, (((1,), (0,)), ((), ())),
                                 preferred_element_type=jnp.float32) + bias
        a2 = jax.lax.dot_general(f2, w, (((1,), (0,)), ((), ())),
                                 preferred_element_type=jnp.float32) + bias
        return 0.5 * (jnp.tanh(a1) + jnp.tanh(a2))

    conf0 = jax.nn.sigmoid(combo(wc_ref[...], bc_ref[...]))
    for j in range(4):
        bp_ref[j] = combo(wb_ref[3 * j:3 * j + 3], bb_ref[j:j + 1]) * 512.0

    iota_l = jax.lax.broadcasted_iota(jnp.int32, (B, NSEL), 1)
    slot_iota = jax.lax.broadcasted_iota(jnp.int32, (B, 128), 1)
    zf = jnp.zeros((B, 128), jnp.float32)
    zi = jnp.zeros((B, 128), jnp.int32)

    def step(k, carry):
        conf, sc, ix = carry
        m = jnp.max(conf, axis=1, keepdims=True)
        sel = jnp.min(jnp.where(conf == m, iota_l, NSEL),
                      axis=1, keepdims=True)
        oh = slot_iota == k
        sc = jnp.where(oh, m, sc)
        ix = jnp.where(oh, sel, ix)
        conf = jnp.where(iota_l == sel, -1.0, conf)
        return conf, sc, ix

    _, sc, ix = jax.lax.fori_loop(0, KEEP, step, (conf0, zf, zi))
    sc_ref[...] = sc[:, :KEEP]
    lb_ref[...] = ix[:, :KEEP] // TOPK_ANCH
    ix_ref[...] = ix


def _headtopk_pallas(f1, f2, Wc, bc, Wb, bb):
    return pl.pallas_call(
        _headtopk_body,
        out_shape=(
            jax.ShapeDtypeStruct((B, KEEP), jnp.float32),
            jax.ShapeDtypeStruct((B, KEEP), jnp.int32),
            jax.ShapeDtypeStruct((B, 128), jnp.int32),
            jax.ShapeDtypeStruct((4, B, NSEL), jnp.float32),
        ),
    )(f1, f2, Wc, bc, Wb, bb)


def _make_sc_gather():
    info = plsc.get_sparse_core_info()
    nc = info.num_cores

    mesh = plsc.VectorSubcoreMesh(core_axis_name="c", subcore_axis_name="s")

    @functools.partial(
        pl.kernel, mesh=mesh,
        out_type=jax.ShapeDtypeStruct((B, 4, KPADDED), jnp.float32),
        compiler_params=pltpu.CompilerParams(needs_layout_passes=False),
        scratch_types=[
            pltpu.VMEM((NSEL,), jnp.float32),
            pltpu.VMEM((NSEL,), jnp.float32),
            pltpu.VMEM((NSEL,), jnp.float32),
            pltpu.VMEM((NSEL,), jnp.float32),
            pltpu.VMEM((128,), jnp.int32),
            pltpu.VMEM((4, KPADDED), jnp.float32),
        ],
    )
    def sc_gather(bp_hbm, ix_hbm, out_hbm, p0, p1, p2, p3, ixv, outv):
        wid = lax.axis_index("s") * nc + lax.axis_index("c")

        @pl.when(wid < B)
        def _():
            planes = [p0, p1, p2, p3]
            for j in range(4):
                pltpu.sync_copy(bp_hbm.at[j, wid], planes[j])
            pltpu.sync_copy(ix_hbm.at[wid], ixv)
            for j in range(4):
                for t in range(KPADDED // 16):
                    idx = ixv[pl.ds(t * 16, 16)]
                    outv[j, pl.ds(t * 16, 16)] = plsc.load_gather(
                        planes[j], [idx])
            pltpu.sync_copy(outv, out_hbm.at[wid])

    return sc_gather


_sc_gather = _make_sc_gather()


def kernel(x, pos, W, b):
    p1 = pos.reshape(HW)
    p2 = pos[::-1, :].reshape(HW)
    P = jnp.stack([p1, p2], axis=0)
    x3 = x.reshape(ROWS, H, W_IMG)
    fp = _feat_pallas(x3, P)
    f1 = fp[:, 0].reshape(B, C)
    f2 = fp[:, 1].reshape(B, C)

    Wt = W.reshape(C, NUM_CLASSES, TOPK_ANCH, 5)
    bt = b.reshape(NUM_CLASSES, TOPK_ANCH, 5)
    Wc = Wt[:, 1:, :, 0].reshape(C, NSEL)
    bc = bt[1:, :, 0].reshape(1, NSEL)
    Wb = jnp.transpose(Wt[:, 1:, :, 1:], (3, 0, 1, 2)).reshape(4 * C, NSEL)
    bb = jnp.transpose(bt[1:, :, 1:], (2, 0, 1)).reshape(4, NSEL)

    top_scores, labels, ix, bp = _headtopk_pallas(f1, f2, Wc, bc, Wb, bb)
    gathered = _sc_gather(bp, ix)
    sel_boxes = jnp.transpose(gathered[:, :, :KEEP], (0, 2, 1))
    return sel_boxes, top_scores, labels


# speedup vs baseline: 1.5268x; 1.2393x over previous
import functools

import jax
import jax.numpy as jnp
from jax import lax
from jax.experimental import pallas as pl
from jax.experimental.pallas import tpu as pltpu, tpu_sc as plsc

B, C, H, W_IMG = 16, 3, 512, 512
NUM_CLASSES, TOPK_ANCH, KEEP = 21, 200, 100
HW = H * W_IMG
ROWS = B * C
RG = 8
NSEL = (NUM_CLASSES - 1) * TOPK_ANCH
KPADDED = 112


def _feat_body(x_ref, p_ref, o_ref):
    xb = x_ref[...].reshape(RG, HW)
    o_ref[...] = jax.lax.dot_general(
        xb, p_ref[...],
        dimension_numbers=(((1,), (1,)), ((), ())),
        preferred_element_type=jnp.float32)


def _feat_pallas(x3, P):
    return pl.pallas_call(
        _feat_body,
        grid=(ROWS // RG,),
        in_specs=[
            pl.BlockSpec((RG, H, W_IMG), lambda i: (i, 0, 0)),
            pl.BlockSpec((2, HW), lambda i: (0, 0)),
        ],
        out_specs=pl.BlockSpec((RG, 2), lambda i: (i, 0)),
        out_shape=jax.ShapeDtypeStruct((ROWS, 2), jnp.float32),
    )(x3, P)


def _headtopk_body(f1_ref, f2_ref, wc_ref, bc_ref, wb_ref, bb_ref,
                   sc_ref, lb_ref, ix_ref, bp_ref):
    f1 = f1_ref[...]
    f2 = f2_ref[...]

    def combo(w, bias):
        a1 = jax.lax.dot_general(f1, w, (((1,), (0,)), ((), ())),
                                 preferred_element_type=jnp.float32) + bias
        a2 = jax.lax.dot_general(f2, w, (((1,), (0,)), ((), ())),
                                 preferred_element_type=jnp.float32) + bias
        return 0.5 * (jnp.tanh(a1) + jnp.tanh(a2))

    conf0 = jax.nn.sigmoid(combo(wc_ref[...], bc_ref[...]))
    for j in range(4):
        bp_ref[j] = combo(wb_ref[3 * j:3 * j + 3], bb_ref[j:j + 1]) * 512.0

    iota_l = jax.lax.broadcasted_iota(jnp.int32, (B, NSEL), 1)
    slot_iota = jax.lax.broadcasted_iota(jnp.int32, (B, 128), 1)
    zf = jnp.zeros((B, 128), jnp.float32)
    zi = jnp.zeros((B, 128), jnp.int32)

    def step(k, carry):
        conf, sc, ix = carry
        m = jnp.max(conf, axis=1, keepdims=True)
        sel = jnp.min(jnp.where(conf == m, iota_l, NSEL),
                      axis=1, keepdims=True)
        oh = slot_iota == k
        sc = jnp.where(oh, m, sc)
        ix = jnp.where(oh, sel, ix)
        conf = jnp.where(iota_l == sel, -1.0, conf)
        return conf, sc, ix

    _, sc, ix = jax.lax.fori_loop(0, KEEP, step, (conf0, zf, zi))
    sc_ref[...] = sc[:, :KEEP]
    lb_ref[...] = ix[:, :KEEP] // TOPK_ANCH
    ix_ref[...] = ix


def _headtopk_pallas(f1, f2, Wc, bc, Wb, bb):
    return pl.pallas_call(
        _headtopk_body,
        out_shape=(
            jax.ShapeDtypeStruct((B, KEEP), jnp.float32),
            jax.ShapeDtypeStruct((B, KEEP), jnp.int32),
            jax.ShapeDtypeStruct((B, 128), jnp.int32),
            jax.ShapeDtypeStruct((4, B, NSEL), jnp.float32),
        ),
    )(f1, f2, Wc, bc, Wb, bb)


def _make_sc_wprep():
    info = plsc.get_sparse_core_info()
    nc = info.num_cores
    mesh = plsc.VectorSubcoreMesh(core_axis_name="c", subcore_axis_name="s")

    @functools.partial(
        pl.kernel, mesh=mesh,
        out_type=jax.ShapeDtypeStruct((20, NSEL), jnp.float32),
        compiler_params=pltpu.CompilerParams(needs_layout_passes=False),
        scratch_types=[
            pltpu.VMEM((NUM_CLASSES * 1000,), jnp.float32),
            pltpu.VMEM((NSEL,), jnp.float32),
        ],
    )
    def sc_wprep(w_hbm, b_hbm, out_hbm, srcv, dstv):
        wid = lax.axis_index("s") * nc + lax.axis_index("c")

        @pl.when(wid < 20)
        def _():
            c = jnp.where(wid < 15, jax.lax.rem(wid - 3, 3), 0)
            c = jnp.where(wid < 3, wid, c)
            field = jnp.where(wid < 3, 0, 0)
            field = jnp.where((wid >= 3) & (wid < 15),
                              (wid - 3) // 3 + 1, field)
            field = jnp.where(wid >= 16, wid - 15, field)
            from_b = wid >= 15

            @pl.when(jnp.logical_not(from_b))
            def _():
                pltpu.sync_copy(w_hbm.at[c], srcv)

            @pl.when(from_b)
            def _():
                pltpu.sync_copy(b_hbm.at[0], srcv)

            def chunk(t, _):
                jv = jax.lax.iota(jnp.int32, 16) + t * 16
                src = (1000 + (jv // TOPK_ANCH) * 1000
                       + jax.lax.rem(jv, TOPK_ANCH) * 5 + field)
                dstv[pl.ds(t * 16, 16)] = plsc.load_gather(srcv, [src])
                return 0

            jax.lax.fori_loop(0, NSEL // 16, chunk, 0)
            pltpu.sync_copy(dstv, out_hbm.at[wid])

    return sc_wprep


_sc_wprep = _make_sc_wprep()


def _make_sc_gather():
    info = plsc.get_sparse_core_info()
    nc = info.num_cores

    mesh = plsc.VectorSubcoreMesh(core_axis_name="c", subcore_axis_name="s")

    @functools.partial(
        pl.kernel, mesh=mesh,
        out_type=jax.ShapeDtypeStruct((B, 4, KPADDED), jnp.float32),
        compiler_params=pltpu.CompilerParams(needs_layout_passes=False),
        scratch_types=[
            pltpu.VMEM((NSEL,), jnp.float32),
            pltpu.VMEM((NSEL,), jnp.float32),
            pltpu.VMEM((NSEL,), jnp.float32),
            pltpu.VMEM((NSEL,), jnp.float32),
            pltpu.VMEM((128,), jnp.int32),
            pltpu.VMEM((4, KPADDED), jnp.float32),
        ],
    )
    def sc_gather(bp_hbm, ix_hbm, out_hbm, p0, p1, p2, p3, ixv, outv):
        wid = lax.axis_index("s") * nc + lax.axis_index("c")

        @pl.when(wid < B)
        def _():
            planes = [p0, p1, p2, p3]
            for j in range(4):
                pltpu.sync_copy(bp_hbm.at[j, wid], planes[j])
            pltpu.sync_copy(ix_hbm.at[wid], ixv)
            for j in range(4):
                for t in range(KPADDED // 16):
                    idx = ixv[pl.ds(t * 16, 16)]
                    outv[j, pl.ds(t * 16, 16)] = plsc.load_gather(
                        planes[j], [idx])
            pltpu.sync_copy(outv, out_hbm.at[wid])

    return sc_gather


_sc_gather = _make_sc_gather()


def kernel(x, pos, W, b):
    p1 = pos.reshape(HW)
    p2 = pos[::-1, :].reshape(HW)
    P = jnp.stack([p1, p2], axis=0)
    x3 = x.reshape(ROWS, H, W_IMG)
    fp = _feat_pallas(x3, P)
    f1 = fp[:, 0].reshape(B, C)
    f2 = fp[:, 1].reshape(B, C)

    wprep = _sc_wprep(W, b.reshape(1, -1))
    Wc = wprep[0:3]
    Wb = wprep[3:15]
    bc = wprep[15:16]
    bb = wprep[16:20]

    top_scores, labels, ix, bp = _headtopk_pallas(f1, f2, Wc, bc, Wb, bb)
    gathered = _sc_gather(bp, ix)
    sel_boxes = jnp.transpose(gathered[:, :, :KEEP], (0, 2, 1))
    return sel_boxes, top_scores, labels
